# baseline (speedup 1.0000x reference)
import jax
import jax.numpy as jnp
from jax.experimental import pallas as pl
from jax.experimental.pallas import tpu as pltpu

ALPHA = 0.2
N = 512
NBLK = 4
BLK = N // NBLK


def _leaky_relu(v):
    return jnp.maximum(v, ALPHA * v)


def _attention(hs, adj_blocks, a_lo, a_hi, waits=None):
    f1s = [jnp.dot(h, a_lo, preferred_element_type=jnp.float32) for h in hs]
    f2ts = [jnp.dot(h, a_hi,
                    preferred_element_type=jnp.float32).reshape(1, -1)
            for h in hs]
    blocks = [[] for _ in hs]
    for k in range(NBLK):
        if waits is not None:
            waits[k]()
        ab = adj_blocks(k)
        for b, (h, f1, f2t) in enumerate(zip(hs, f1s, f2ts)):
            p = ab * jnp.exp(_leaky_relu(f1[k * BLK:(k + 1) * BLK] + f2t))
            s = jnp.sum(p, axis=1, keepdims=True)
            num = jnp.dot(p, h, preferred_element_type=jnp.float32)
            blocks[b].append(num / s)
    return [jnp.concatenate(bs, axis=0) for bs in blocks]


def _gatcell_kernel(x_ref, adj_ref, w1_ref, a1_ref, w2_ref, a2_ref, out_ref,
                    adj_vmem, adj_sem):
    copies = [
        pltpu.make_async_copy(
            adj_ref.at[pl.ds(k * BLK, BLK), :],
            adj_vmem.at[pl.ds(k * BLK, BLK), :],
            adj_sem.at[k])
        for k in range(NBLK)
    ]
    for c in copies:
        c.start()

    xs = [x_ref[b] for b in range(x_ref.shape[0])]

    w1eff = w1_ref[:64, :] + w1_ref[64:, :]
    h1s = [jnp.dot(x, w1eff, preferred_element_type=jnp.float32) for x in xs]
    gvs = _attention(h1s, lambda k: adj_vmem[k * BLK:(k + 1) * BLK, :],
                     a1_ref[:128, :], a1_ref[128:, :],
                     waits=[c.wait for c in copies])

    rs_zs = [(jax.nn.sigmoid(gv[:, :64]), jax.nn.sigmoid(gv[:, 64:]))
             for gv in gvs]
    h2s = [jnp.dot(x, w2_ref[:64, :], preferred_element_type=jnp.float32)
           + jnp.dot(r * x, w2_ref[64:, :], preferred_element_type=jnp.float32)
           for x, (r, _) in zip(xs, rs_zs)]
    hps = _attention(h2s, lambda k: adj_vmem[k * BLK:(k + 1) * BLK, :],
                     a2_ref[:64, :], a2_ref[64:, :])

    for b, (x, (_, z), hp) in enumerate(zip(xs, rs_zs, hps)):
        t = jnp.tanh(hp)
        out_ref[b] = t + z * (x - t)


def kernel(X, adj, W1, a1, W2, a2):
    return pl.pallas_call(
        _gatcell_kernel,
        in_specs=[
            pl.BlockSpec(memory_space=pltpu.MemorySpace.VMEM),
            pl.BlockSpec(memory_space=pl.ANY),
            pl.BlockSpec(memory_space=pltpu.MemorySpace.VMEM),
            pl.BlockSpec(memory_space=pltpu.MemorySpace.VMEM),
            pl.BlockSpec(memory_space=pltpu.MemorySpace.VMEM),
            pl.BlockSpec(memory_space=pltpu.MemorySpace.VMEM),
        ],
        scratch_shapes=[
            pltpu.VMEM((N, N), jnp.float32),
            pltpu.SemaphoreType.DMA((NBLK,)),
        ],
        out_shape=jax.ShapeDtypeStruct(X.shape, X.dtype),
    )(X, adj, W1, a1, W2, a2)
